# baseline (speedup 1.0000x reference)
import functools

import jax
import jax.numpy as jnp
from jax import lax
from jax.experimental import pallas as pl
from jax.experimental.pallas import tpu as pltpu
from jax.experimental.pallas import tpu_sc as plsc

NV = 50000
NC = 200000
NE = 800000
D = 128
QM = 32

NE_PAD = 819200
IDX_ROWS = NE_PAD // 128
ROWS_PER_TILE = IDX_ROWS // 16
KGRP = 2
NGRP = ROWS_PER_TILE // KGRP
CH = 53504
CSLOTS = KGRP * 128 + 16
NSENT = 128
W = 32


def _take16(x, idx):
    return lax.gather(
        x, idx[:, None],
        dimension_numbers=lax.GatherDimensionNumbers(
            offset_dims=(), collapsed_slice_dims=(0,), start_index_map=(0,)),
        slice_sizes=(1,), mode=lax.GatherScatterMode.PROMISE_IN_BOUNDS)


def _make_sc_pass(*, row_split, src_off_per_core, n_rounds, out_rows):
    mesh = plsc.VectorSubcoreMesh(core_axis_name="c", subcore_axis_name="s")
    chunk_rows = CH + NSENT
    zrows_per_tile = chunk_rows // 16
    crows_per_tile = CH // 16

    def body(table_hbm, src2d_hbm, dst2d_hbm, zeros_hbm, out_hbm,
             chunk, sidx_a, didx_a, csi_a, csl_a, csl2_a, rows_a,
             sidx_b, didx_b, csi_b, csl_b, csl2_b, rows_b,
             sem_i, sem_ga, sem_gb, sem_sa, sem_sb):
        c = lax.axis_index("c")
        s = lax.axis_index("s")
        lane = lax.iota(jnp.int32, 16)
        tile_row0 = s * ROWS_PER_TILE
        bufs = ((sidx_a, didx_a, csi_a, csl_a, csl2_a, rows_a, sem_ga, sem_sa),
                (sidx_b, didx_b, csi_b, csl_b, csl2_b, rows_b, sem_gb, sem_sb))

        def prep(g, buf, lo):
            sidx, didx, csi_f, csl_f, csl2, _, _, _ = buf
            row0 = tile_row0 + g * KGRP
            cp_s = pltpu.async_copy(src2d_hbm.at[pl.ds(row0, KGRP)], sidx, sem_i)
            cp_d = pltpu.async_copy(dst2d_hbm.at[pl.ds(row0, KGRP)], didx, sem_i)
            cp_s.wait()
            cp_d.wait()
            cnt = jnp.int32(0)
            for j in range(KGRP):
                for l in range(8):
                    dv = didx[j, pl.ds(l * 16, 16)]
                    sv = sidx[j, pl.ds(l * 16, 16)]
                    if not row_split and src_off_per_core:
                        sv = sv + c * src_off_per_core
                    m = (dv >= lo) & (dv < lo + CH)
                    key = jnp.where(m, lane, 16 + lane)
                    dummy = s * 1024 + (j * 8 + l) * 16 + lane
                    sent = CH + ((lane + 16 * (j * 8 + l)) % NSENT)
                    perm = plsc.sort_key_val(key, lane)[1]
                    csi_f[pl.ds(cnt, 16)] = _take16(jnp.where(m, sv, dummy), perm)
                    csl_f[pl.ds(cnt, 16)] = _take16(jnp.where(m, dv - lo, sent), perm)
                    cnt = cnt + jnp.sum(m.astype(jnp.int32))
            ceilb = ((cnt + 127) // 128) * 128
            for i in range(8):
                @pl.when(cnt + 16 * i < ceilb)
                def _(i=i):
                    csi_f[pl.ds(cnt + 16 * i, 16)] = s * 1024 + 16 * i + lane
                    csl_f[pl.ds(cnt + 16 * i, 16)] = CH + ((lane + 16 * i) % NSENT)
            for j in range(KGRP):
                for i in range(8):
                    csl2[j, pl.ds(i * 16, 16)] = csl_f[pl.ds(j * 128 + i * 16, 16)]
            return cnt

        def fire_gather(buf, cnt):
            _, _, csi_f, _, _, rows, sem_g, _ = buf
            for j in range(KGRP):
                @pl.when(cnt > j * 128)
                def _(j=j):
                    pltpu.async_copy(table_hbm.at[csi_f.at[pl.ds(j * 128, 128)]],
                                     rows.at[pl.ds(j * 128, 128)], sem_g)

        def drain_gather(buf, cnt):
            _, _, csi_f, _, _, rows, sem_g, _ = buf
            for j in range(KGRP):
                @pl.when(cnt > j * 128)
                def _(j=j):
                    pltpu.make_async_copy(
                        table_hbm.at[csi_f.at[pl.ds(j * 128, 128)]],
                        rows.at[pl.ds(j * 128, 128)], sem_g).wait()

        def fire_scatter(buf, cnt):
            _, _, _, _, csl2, rows, _, sem_s = buf
            for j in range(KGRP):
                @pl.when(cnt > j * 128)
                def _(j=j):
                    pltpu.async_copy(rows.at[pl.ds(j * 128, 128)],
                                     chunk.at[csl2.at[j]], sem_s, add=True)

        def drain_scatter(buf, cnt):
            _, _, _, _, csl2, rows, _, sem_s = buf
            for j in range(KGRP):
                @pl.when(cnt > j * 128)
                def _(j=j):
                    pltpu.make_async_copy(rows.at[pl.ds(j * 128, 128)],
                                          chunk.at[csl2.at[j]], sem_s).wait()

        for r in range(n_rounds):
            if row_split:
                lo = (r * 2 + c) * CH
                out_base = lo
            else:
                lo = r * CH
                out_base = r * CH
            pltpu.sync_copy(zeros_hbm.at[pl.ds(s * zrows_per_tile, zrows_per_tile)],
                            chunk.at[pl.ds(s * zrows_per_tile, zrows_per_tile)])
            plsc.subcore_barrier()

            cnt_a0 = prep(0, bufs[0], lo)
            fire_gather(bufs[0], cnt_a0)

            def pair(t, cnt_a):
                a, b = bufs
                cnt_b = prep(2 * t + 1, b, lo)
                drain_gather(a, cnt_a)
                fire_scatter(a, cnt_a)
                fire_gather(b, cnt_b)
                drain_scatter(a, cnt_a)
                g_next = jnp.where(2 * t + 2 < NGRP, 2 * t + 2, 0)
                cnt_a2 = prep(g_next, a, lo)
                drain_gather(b, cnt_b)
                fire_scatter(b, cnt_b)
                fire_gather(a, cnt_a2)
                drain_scatter(b, cnt_b)
                return cnt_a2

            cnt_fin = lax.fori_loop(0, NGRP // 2, pair, cnt_a0)
            drain_gather(bufs[0], cnt_fin)
            plsc.subcore_barrier()
            if row_split:
                pltpu.sync_copy(chunk.at[pl.ds(s * crows_per_tile, crows_per_tile)],
                                out_hbm.at[pl.ds(out_base + s * crows_per_tile,
                                                 crows_per_tile)])
            else:
                pltpu.sync_copy(chunk.at[pl.ds(s * crows_per_tile, crows_per_tile)],
                                out_hbm.at[c].at[pl.ds(out_base + s * crows_per_tile,
                                                       crows_per_tile)])
            plsc.subcore_barrier()

    if row_split:
        out_type = jax.ShapeDtypeStruct((out_rows, W), jnp.float32)
    else:
        out_type = jax.ShapeDtypeStruct((2, out_rows, W), jnp.float32)

    return pl.kernel(
        body,
        out_type=out_type,
        mesh=mesh,
        compiler_params=pltpu.CompilerParams(use_tc_tiling_on_sc=False,
                                             needs_layout_passes=False),
        scratch_types=(
            [pltpu.VMEM_SHARED((chunk_rows, W), jnp.float32)]
            + 2 * [pltpu.VMEM((KGRP, 128), jnp.int32),
                   pltpu.VMEM((KGRP, 128), jnp.int32),
                   pltpu.VMEM((CSLOTS,), jnp.int32),
                   pltpu.VMEM((CSLOTS,), jnp.int32),
                   pltpu.VMEM((KGRP, 128), jnp.int32),
                   pltpu.VMEM((KGRP * 128, W), jnp.float32)]
            + 5 * [pltpu.SemaphoreType.DMA]
        ),
    )


def _softplus(x):
    return jnp.maximum(x, 0.0) + jnp.log1p(jnp.exp(-jnp.abs(x)))


def _q_kernel(vx_ref, nz_ref, w1a_ref, w1b_ref, b1_ref, w2_ref, b2_ref,
              q_ref, lit_ref):
    h = (jnp.dot(vx_ref[...], w1a_ref[...], preferred_element_type=jnp.float32)
         + jnp.dot(nz_ref[...], w1b_ref[...], preferred_element_type=jnp.float32)
         + b1_ref[...])
    h = jnp.maximum(h, 0.0)
    q = jnp.dot(h, w2_ref[...], preferred_element_type=jnp.float32) + b2_ref[...]
    q_ref[...] = q
    lit_ref[0, :, :] = _softplus(q)
    lit_ref[1, :, :] = _softplus(-q)


def _clause_kernel(cx_ref, csum_ref, w1_ref, b1_ref, w2_ref, b2_ref,
                   g_ref, val_ref, colsum_ref, sq_ref):
    i = pl.program_id(0)
    cl = jnp.exp(-csum_ref[...])
    unit = jnp.concatenate([cx_ref[...], 4.0 * cl], axis=-1)
    h = jnp.dot(unit, w1_ref[...], preferred_element_type=jnp.float32) + b1_ref[...]
    h = jnp.maximum(h, 0.0)
    d = jnp.dot(h, w2_ref[...], preferred_element_type=jnp.float32) + b2_ref[...]
    g_ref[0, :, :] = -cl
    g_ref[1, :, :] = d[:, :QM]
    val = d[:, QM:]
    val_ref[...] = val
    bs = jnp.sum(val, axis=0, keepdims=True)
    bq = jnp.sum(val * val, axis=0, keepdims=True)

    @pl.when(i == 0)
    def _():
        colsum_ref[...] = bs
        sq_ref[...] = bq

    @pl.when(i != 0)
    def _():
        colsum_ref[...] += bs
        sq_ref[...] += bq


def _pairnorm_apply_kernel(val_ref, res_ref, colsum_ref, sq_ref, o_ref, *, n):
    mu = colsum_ref[...] / n
    var = jnp.sum(sq_ref[...]) / (n * D) - jnp.sum(mu * mu) / D
    scale = lax.rsqrt(var + 1e-6)
    o_ref[...] = (val_ref[...] - mu) * scale + 0.1 * res_ref[...]


def _update_kernel(q_ref, glt_ref, glb_ref, dw_ref, vx_ref, vl_ref, ldw_ref,
                   w1_ref, b1_ref, w2_ref, b2_ref, w3_ref, b3_ref,
                   val_ref, colsum_ref, sq_ref):
    i = pl.program_id(0)
    q = q_ref[...]
    sig_p = jax.nn.sigmoid(q)
    sig_n = jax.nn.sigmoid(-q)
    vg = (sig_p * glt_ref[...] - sig_n * glb_ref[...]) * dw_ref[...]
    vl = vl_ref[...] * ldw_ref[...]
    unit = jnp.concatenate([vg, vx_ref[...], vl], axis=-1)
    h = jnp.dot(unit, w1_ref[...], preferred_element_type=jnp.float32) + b1_ref[...]
    h = jnp.maximum(h, 0.0)
    h = jnp.dot(h, w2_ref[...], preferred_element_type=jnp.float32) + b2_ref[...]
    h = jnp.maximum(h, 0.0)
    val = jnp.dot(h, w3_ref[...], preferred_element_type=jnp.float32) + b3_ref[...]
    val_ref[...] = val
    bs = jnp.sum(val, axis=0, keepdims=True)
    bq = jnp.sum(val * val, axis=0, keepdims=True)

    @pl.when(i == 0)
    def _():
        colsum_ref[...] = bs
        sq_ref[...] = bq

    @pl.when(i != 0)
    def _():
        colsum_ref[...] += bs
        sq_ref[...] += bq


def _row_spec(b, w):
    return pl.BlockSpec((b, w), lambda i: (i, 0))


def _full_spec(shape):
    return pl.BlockSpec(shape, lambda i: tuple(0 for _ in shape))


def kernel(variable_x, clause_x, noise, degree_weight, literal_degree_weight,
           clause_edge_index, literal_edge_index,
           query_params, clause_params, update_params):
    f32 = jnp.float32
    qw1, qb1, qw2, qb2 = query_params
    cw1, cb1, cw2, cb2 = clause_params
    uw1, ub1, uw2, ub2, uw3, ub3 = update_params

    BQ = 1000
    q, lit2 = pl.pallas_call(
        _q_kernel,
        grid=(NV // BQ,),
        in_specs=[_row_spec(BQ, D), _row_spec(BQ, 4),
                  pl.BlockSpec((D, qw1.shape[1]), lambda i: (0, 0)),
                  pl.BlockSpec((4, qw1.shape[1]), lambda i: (0, 0)),
                  pl.BlockSpec((1, qb1.shape[0]), lambda i: (0, 0)),
                  pl.BlockSpec(qw2.shape, lambda i: (0, 0)),
                  pl.BlockSpec((1, QM), lambda i: (0, 0))],
        out_specs=[_row_spec(BQ, QM),
                   pl.BlockSpec((2, BQ, QM), lambda i: (0, i, 0))],
        out_shape=[jax.ShapeDtypeStruct((NV, QM), f32),
                   jax.ShapeDtypeStruct((2, NV, QM), f32)],
    )(variable_x, noise, qw1[:D], qw1[D:], qb1[None, :], qw2, qb2[None, :])
    lit = lit2.reshape(2 * NV, QM)

    pad = NE_PAD - NE
    le2d = jnp.concatenate([literal_edge_index,
                            jnp.zeros((pad,), jnp.int32)]).reshape(IDX_ROWS, 128)
    ce2d = jnp.concatenate([clause_edge_index,
                            jnp.full((pad,), -1, jnp.int32)]).reshape(IDX_ROWS, 128)
    ce2d_src = jnp.concatenate([clause_edge_index,
                                jnp.zeros((pad,), jnp.int32)]).reshape(IDX_ROWS, 128)
    le2d_dst = jnp.concatenate([literal_edge_index,
                                jnp.full((pad,), -1, jnp.int32)]).reshape(IDX_ROWS, 128)
    zeros_chunk = jnp.zeros((CH + NSENT, W), f32)

    p1_rounds = 2
    p1_out_rows = p1_rounds * 2 * CH
    sc_pass1 = _make_sc_pass(row_split=True, src_off_per_core=0,
                             n_rounds=p1_rounds, out_rows=p1_out_rows)
    csum_pad = sc_pass1(lit, le2d, ce2d, zeros_chunk)
    csum = csum_pad[:NC]

    BC = 2000
    H1 = cw1.shape[1]
    g2, cval, ccolsum, csq = pl.pallas_call(
        _clause_kernel,
        grid=(NC // BC,),
        in_specs=[_row_spec(BC, D), _row_spec(BC, QM),
                  pl.BlockSpec(cw1.shape, lambda i: (0, 0)),
                  pl.BlockSpec((1, H1), lambda i: (0, 0)),
                  pl.BlockSpec(cw2.shape, lambda i: (0, 0)),
                  pl.BlockSpec((1, cw2.shape[1]), lambda i: (0, 0))],
        out_specs=[pl.BlockSpec((2, BC, QM), lambda i: (0, i, 0)),
                   _row_spec(BC, D),
                   pl.BlockSpec((1, D), lambda i: (0, 0)),
                   pl.BlockSpec((1, D), lambda i: (0, 0))],
        out_shape=[jax.ShapeDtypeStruct((2, NC, QM), f32),
                   jax.ShapeDtypeStruct((NC, D), f32),
                   jax.ShapeDtypeStruct((1, D), f32),
                   jax.ShapeDtypeStruct((1, D), f32)],
    )(clause_x, csum, cw1, cb1[None, :], cw2, cb2[None, :])
    gcat = g2.reshape(2 * NC, QM)

    p2_rounds = 2
    p2_out_rows = p2_rounds * CH
    sc_pass2 = _make_sc_pass(row_split=False, src_off_per_core=NC,
                             n_rounds=p2_rounds, out_rows=p2_out_rows)
    acc2 = sc_pass2(gcat, ce2d_src, le2d_dst, zeros_chunk)
    g_lit = acc2[0, :2 * NV]
    vl_pre = acc2[1, :2 * NV]

    new_clause_x = pl.pallas_call(
        functools.partial(_pairnorm_apply_kernel, n=float(NC)),
        grid=(NC // BC,),
        in_specs=[_row_spec(BC, D), _row_spec(BC, D),
                  pl.BlockSpec((1, D), lambda i: (0, 0)),
                  pl.BlockSpec((1, D), lambda i: (0, 0))],
        out_specs=_row_spec(BC, D),
        out_shape=jax.ShapeDtypeStruct((NC, D), f32),
    )(cval, clause_x, ccolsum, csq)

    ldw64 = jnp.repeat(literal_degree_weight.reshape(NV, 2), QM, axis=1)
    vl64 = vl_pre.reshape(NV, 2 * QM)
    dw32 = jnp.repeat(degree_weight, QM, axis=1)
    BU = 1000
    H2 = uw1.shape[1]
    uval, ucolsum, usq = pl.pallas_call(
        _update_kernel,
        grid=(NV // BU,),
        in_specs=[_row_spec(BU, QM), _row_spec(BU, QM), _row_spec(BU, QM),
                  _row_spec(BU, QM), _row_spec(BU, D), _row_spec(BU, 2 * QM),
                  _row_spec(BU, 2 * QM),
                  pl.BlockSpec(uw1.shape, lambda i: (0, 0)),
                  pl.BlockSpec((1, H2), lambda i: (0, 0)),
                  pl.BlockSpec(uw2.shape, lambda i: (0, 0)),
                  pl.BlockSpec((1, H2), lambda i: (0, 0)),
                  pl.BlockSpec(uw3.shape, lambda i: (0, 0)),
                  pl.BlockSpec((1, D), lambda i: (0, 0))],
        out_specs=[_row_spec(BU, D),
                   pl.BlockSpec((1, D), lambda i: (0, 0)),
                   pl.BlockSpec((1, D), lambda i: (0, 0))],
        out_shape=[jax.ShapeDtypeStruct((NV, D), f32),
                   jax.ShapeDtypeStruct((1, D), f32),
                   jax.ShapeDtypeStruct((1, D), f32)],
    )(q, g_lit[:NV], g_lit[NV:], dw32, variable_x, vl64, ldw64,
      uw1, ub1[None, :], uw2, ub2[None, :], uw3, ub3[None, :])

    new_variable_x = pl.pallas_call(
        functools.partial(_pairnorm_apply_kernel, n=float(NV)),
        grid=(NV // BU,),
        in_specs=[_row_spec(BU, D), _row_spec(BU, D),
                  pl.BlockSpec((1, D), lambda i: (0, 0)),
                  pl.BlockSpec((1, D), lambda i: (0, 0))],
        out_specs=_row_spec(BU, D),
        out_shape=jax.ShapeDtypeStruct((NV, D), f32),
    )(uval, variable_x, ucolsum, usq)

    return new_variable_x, new_clause_x


# speedup vs baseline: 1.0116x; 1.0116x over previous
import functools

import jax
import jax.numpy as jnp
from jax import lax
from jax.experimental import pallas as pl
from jax.experimental.pallas import tpu as pltpu
from jax.experimental.pallas import tpu_sc as plsc

NV = 50000
NC = 200000
NE = 800000
D = 128
QM = 32

NE_PAD = 819200
IDX_ROWS = NE_PAD // 128
ROWS_PER_TILE = IDX_ROWS // 16
KGRP = 2
NGRP = ROWS_PER_TILE // KGRP
CH = 53504
CSLOTS = KGRP * 128 + 16
NSENT = 128
W = 32


def _take16(x, idx):
    return lax.gather(
        x, idx[:, None],
        dimension_numbers=lax.GatherDimensionNumbers(
            offset_dims=(), collapsed_slice_dims=(0,), start_index_map=(0,)),
        slice_sizes=(1,), mode=lax.GatherScatterMode.PROMISE_IN_BOUNDS)


def _make_sc_pass(*, row_split, src_off_per_core, n_rounds, out_rows):
    mesh = plsc.VectorSubcoreMesh(core_axis_name="c", subcore_axis_name="s")
    chunk_rows = CH + NSENT
    zrows_per_tile = chunk_rows // 16
    crows_per_tile = CH // 16

    def body(table_hbm, src2d_hbm, dst2d_hbm, zeros_hbm, out_hbm,
             chunk, sidx_a, didx_a, csi_a, csl_a, csl2_a, rows_a,
             sidx_b, didx_b, csi_b, csl_b, csl2_b, rows_b,
             sem_i, sem_ga, sem_gb, sem_sa, sem_sb):
        c = lax.axis_index("c")
        s = lax.axis_index("s")
        lane = lax.iota(jnp.int32, 16)
        tile_row0 = s * ROWS_PER_TILE
        bufs = ((sidx_a, didx_a, csi_a, csl_a, csl2_a, rows_a, sem_ga, sem_sa),
                (sidx_b, didx_b, csi_b, csl_b, csl2_b, rows_b, sem_gb, sem_sb))

        def prep(g, buf, lo):
            sidx, didx, csi_f, csl_f, csl2, _, _, _ = buf
            row0 = tile_row0 + g * KGRP
            cp_s = pltpu.async_copy(src2d_hbm.at[pl.ds(row0, KGRP)], sidx, sem_i)
            cp_d = pltpu.async_copy(dst2d_hbm.at[pl.ds(row0, KGRP)], didx, sem_i)
            cp_s.wait()
            cp_d.wait()
            cnt = jnp.int32(0)
            for j in range(KGRP):
                for l in range(8):
                    dv = didx[j, pl.ds(l * 16, 16)]
                    sv = sidx[j, pl.ds(l * 16, 16)]
                    if not row_split and src_off_per_core:
                        sv = sv + c * src_off_per_core
                    m = (dv >= lo) & (dv < lo + CH)
                    key = jnp.where(m, lane, 16 + lane)
                    dummy = s * 1024 + (j * 8 + l) * 16 + lane
                    sent = CH + ((lane + 16 * (j * 8 + l)) % NSENT)
                    perm = plsc.sort_key_val(key, lane)[1]
                    csi_f[pl.ds(cnt, 16)] = _take16(jnp.where(m, sv, dummy), perm)
                    csl_f[pl.ds(cnt, 16)] = _take16(jnp.where(m, dv - lo, sent), perm)
                    cnt = cnt + jnp.sum(m.astype(jnp.int32))
            ceilb = ((cnt + 127) // 128) * 128
            for i in range(8):
                @pl.when(cnt + 16 * i < ceilb)
                def _(i=i):
                    csi_f[pl.ds(cnt + 16 * i, 16)] = s * 1024 + 16 * i + lane
                    csl_f[pl.ds(cnt + 16 * i, 16)] = CH + ((lane + 16 * i) % NSENT)
            for j in range(KGRP):
                for i in range(8):
                    csl2[j, pl.ds(i * 16, 16)] = csl_f[pl.ds(j * 128 + i * 16, 16)]
            return cnt

        def fire_gather(buf, cnt):
            _, _, csi_f, _, _, rows, sem_g, _ = buf
            for j in range(KGRP):
                @pl.when(cnt > j * 128)
                def _(j=j):
                    pltpu.async_copy(table_hbm.at[csi_f.at[pl.ds(j * 128, 128)]],
                                     rows.at[pl.ds(j * 128, 128)], sem_g)

        def drain_gather(buf, cnt):
            _, _, csi_f, _, _, rows, sem_g, _ = buf
            for j in range(KGRP):
                @pl.when(cnt > j * 128)
                def _(j=j):
                    pltpu.make_async_copy(
                        table_hbm.at[csi_f.at[pl.ds(j * 128, 128)]],
                        rows.at[pl.ds(j * 128, 128)], sem_g).wait()

        def fire_scatter(buf, cnt):
            _, _, _, _, csl2, rows, _, sem_s = buf
            for j in range(KGRP):
                @pl.when(cnt > j * 128)
                def _(j=j):
                    pltpu.async_copy(rows.at[pl.ds(j * 128, 128)],
                                     chunk.at[csl2.at[j]], sem_s, add=True)

        def drain_scatter(buf, cnt):
            _, _, _, _, csl2, rows, _, sem_s = buf
            for j in range(KGRP):
                @pl.when(cnt > j * 128)
                def _(j=j):
                    pltpu.make_async_copy(rows.at[pl.ds(j * 128, 128)],
                                          chunk.at[csl2.at[j]], sem_s).wait()

        for r in range(n_rounds):
            if row_split:
                lo = (r * 2 + c) * CH
                out_base = lo
            else:
                lo = r * CH
                out_base = r * CH
            pltpu.sync_copy(zeros_hbm.at[pl.ds(s * zrows_per_tile, zrows_per_tile)],
                            chunk.at[pl.ds(s * zrows_per_tile, zrows_per_tile)])
            plsc.subcore_barrier()

            cnt_a0 = prep(0, bufs[0], lo)
            fire_gather(bufs[0], cnt_a0)

            def pair(t, cnt_a):
                a, b = bufs
                cnt_b = prep(2 * t + 1, b, lo)
                drain_gather(a, cnt_a)
                fire_scatter(a, cnt_a)
                fire_gather(b, cnt_b)
                drain_scatter(a, cnt_a)
                g_next = jnp.where(2 * t + 2 < NGRP, 2 * t + 2, 0)
                cnt_a2 = prep(g_next, a, lo)
                drain_gather(b, cnt_b)
                fire_scatter(b, cnt_b)
                fire_gather(a, cnt_a2)
                drain_scatter(b, cnt_b)
                return cnt_a2

            cnt_fin = lax.fori_loop(0, NGRP // 2, pair, cnt_a0)
            drain_gather(bufs[0], cnt_fin)
            plsc.subcore_barrier()
            if row_split:
                pltpu.sync_copy(chunk.at[pl.ds(s * crows_per_tile, crows_per_tile)],
                                out_hbm.at[pl.ds(out_base + s * crows_per_tile,
                                                 crows_per_tile)])
            else:
                pltpu.sync_copy(chunk.at[pl.ds(s * crows_per_tile, crows_per_tile)],
                                out_hbm.at[c].at[pl.ds(out_base + s * crows_per_tile,
                                                       crows_per_tile)])
            plsc.subcore_barrier()

    if row_split:
        out_type = jax.ShapeDtypeStruct((out_rows, W), jnp.float32)
    else:
        out_type = jax.ShapeDtypeStruct((2, out_rows, W), jnp.float32)

    return pl.kernel(
        body,
        out_type=out_type,
        mesh=mesh,
        compiler_params=pltpu.CompilerParams(use_tc_tiling_on_sc=False,
                                             needs_layout_passes=False),
        scratch_types=(
            [pltpu.VMEM_SHARED((chunk_rows, W), jnp.float32)]
            + 2 * [pltpu.VMEM((KGRP, 128), jnp.int32),
                   pltpu.VMEM((KGRP, 128), jnp.int32),
                   pltpu.VMEM((CSLOTS,), jnp.int32),
                   pltpu.VMEM((CSLOTS,), jnp.int32),
                   pltpu.VMEM((KGRP, 128), jnp.int32),
                   pltpu.VMEM((KGRP * 128, W), jnp.float32)]
            + 5 * [pltpu.SemaphoreType.DMA]
        ),
    )


def _softplus(x):
    return jnp.maximum(x, 0.0) + jnp.log1p(jnp.exp(-jnp.abs(x)))


def _q_kernel(vx_ref, nz_ref, w1a_ref, w1b_ref, b1_ref, w2_ref, b2_ref,
              q_ref, lit_ref):
    h = (jnp.dot(vx_ref[...], w1a_ref[...], preferred_element_type=jnp.float32)
         + jnp.dot(nz_ref[...], w1b_ref[...], preferred_element_type=jnp.float32)
         + b1_ref[...])
    h = jnp.maximum(h, 0.0)
    q = jnp.dot(h, w2_ref[...], preferred_element_type=jnp.float32) + b2_ref[...]
    q_ref[...] = q
    lit_ref[0, :, :] = _softplus(q)
    lit_ref[1, :, :] = _softplus(-q)


def _clause_kernel(cx_ref, csum_ref, w1_ref, b1_ref, w2_ref, b2_ref,
                   g_ref, val_ref, colsum_ref, sq_ref):
    i = pl.program_id(0)
    cl = jnp.exp(-csum_ref[...])
    unit = jnp.concatenate([cx_ref[...], 4.0 * cl], axis=-1)
    h = jnp.dot(unit.astype(jnp.bfloat16), w1_ref[...].astype(jnp.bfloat16),
                preferred_element_type=jnp.float32) + b1_ref[...]
    h = jnp.maximum(h, 0.0)
    d = jnp.dot(h.astype(jnp.bfloat16), w2_ref[...].astype(jnp.bfloat16),
                preferred_element_type=jnp.float32) + b2_ref[...]
    g_ref[0, :, :] = -cl
    g_ref[1, :, :] = d[:, :QM]
    val = d[:, QM:]
    val_ref[...] = val.astype(val_ref.dtype)
    bs = jnp.sum(val, axis=0, keepdims=True)
    bq = jnp.sum(val * val, axis=0, keepdims=True)

    @pl.when(i == 0)
    def _():
        colsum_ref[...] = bs
        sq_ref[...] = bq

    @pl.when(i != 0)
    def _():
        colsum_ref[...] += bs
        sq_ref[...] += bq


def _pairnorm_apply_kernel(val_ref, res_ref, colsum_ref, sq_ref, o_ref, *, n):
    mu = colsum_ref[...] / n
    var = jnp.sum(sq_ref[...]) / (n * D) - jnp.sum(mu * mu) / D
    scale = lax.rsqrt(var + 1e-6)
    o_ref[...] = (val_ref[...].astype(jnp.float32) - mu) * scale + 0.1 * res_ref[...]


def _update_kernel(q_ref, glt_ref, glb_ref, dw_ref, vx_ref, vl_ref, ldw_ref,
                   w1_ref, b1_ref, w2_ref, b2_ref, w3_ref, b3_ref,
                   val_ref, colsum_ref, sq_ref):
    i = pl.program_id(0)
    q = q_ref[...]
    sig_p = jax.nn.sigmoid(q)
    sig_n = jax.nn.sigmoid(-q)
    vg = (sig_p * glt_ref[...] - sig_n * glb_ref[...]) * dw_ref[...]
    vl = vl_ref[...] * ldw_ref[...]
    unit = jnp.concatenate([vg, vx_ref[...], vl], axis=-1)
    h = jnp.dot(unit.astype(jnp.bfloat16), w1_ref[...].astype(jnp.bfloat16),
                preferred_element_type=jnp.float32) + b1_ref[...]
    h = jnp.maximum(h, 0.0)
    h = jnp.dot(h.astype(jnp.bfloat16), w2_ref[...].astype(jnp.bfloat16),
                preferred_element_type=jnp.float32) + b2_ref[...]
    h = jnp.maximum(h, 0.0)
    val = jnp.dot(h.astype(jnp.bfloat16), w3_ref[...].astype(jnp.bfloat16),
                  preferred_element_type=jnp.float32) + b3_ref[...]
    val_ref[...] = val
    bs = jnp.sum(val, axis=0, keepdims=True)
    bq = jnp.sum(val * val, axis=0, keepdims=True)

    @pl.when(i == 0)
    def _():
        colsum_ref[...] = bs
        sq_ref[...] = bq

    @pl.when(i != 0)
    def _():
        colsum_ref[...] += bs
        sq_ref[...] += bq


def _row_spec(b, w):
    return pl.BlockSpec((b, w), lambda i: (i, 0))


def _full_spec(shape):
    return pl.BlockSpec(shape, lambda i: tuple(0 for _ in shape))


def kernel(variable_x, clause_x, noise, degree_weight, literal_degree_weight,
           clause_edge_index, literal_edge_index,
           query_params, clause_params, update_params):
    f32 = jnp.float32
    qw1, qb1, qw2, qb2 = query_params
    cw1, cb1, cw2, cb2 = clause_params
    uw1, ub1, uw2, ub2, uw3, ub3 = update_params

    BQ = 1000
    q, lit2 = pl.pallas_call(
        _q_kernel,
        grid=(NV // BQ,),
        in_specs=[_row_spec(BQ, D), _row_spec(BQ, 4),
                  pl.BlockSpec((D, qw1.shape[1]), lambda i: (0, 0)),
                  pl.BlockSpec((4, qw1.shape[1]), lambda i: (0, 0)),
                  pl.BlockSpec((1, qb1.shape[0]), lambda i: (0, 0)),
                  pl.BlockSpec(qw2.shape, lambda i: (0, 0)),
                  pl.BlockSpec((1, QM), lambda i: (0, 0))],
        out_specs=[_row_spec(BQ, QM),
                   pl.BlockSpec((2, BQ, QM), lambda i: (0, i, 0))],
        out_shape=[jax.ShapeDtypeStruct((NV, QM), f32),
                   jax.ShapeDtypeStruct((2, NV, QM), f32)],
    )(variable_x, noise, qw1[:D], qw1[D:], qb1[None, :], qw2, qb2[None, :])
    lit = lit2.reshape(2 * NV, QM)

    pad = NE_PAD - NE
    le2d = jnp.concatenate([literal_edge_index,
                            jnp.full((pad,), -1, jnp.int32)]).reshape(IDX_ROWS, 128)
    ce2d = jnp.concatenate([clause_edge_index,
                            jnp.full((pad,), -1, jnp.int32)]).reshape(IDX_ROWS, 128)
    zeros_chunk = jnp.zeros((CH + NSENT, W), f32)

    p1_rounds = 2
    p1_out_rows = p1_rounds * 2 * CH
    sc_pass1 = _make_sc_pass(row_split=True, src_off_per_core=0,
                             n_rounds=p1_rounds, out_rows=p1_out_rows)
    csum_pad = sc_pass1(lit, le2d, ce2d, zeros_chunk)
    csum = csum_pad[:NC]

    BC = 2000
    H1 = cw1.shape[1]
    g2, cval, ccolsum, csq = pl.pallas_call(
        _clause_kernel,
        grid=(NC // BC,),
        in_specs=[_row_spec(BC, D), _row_spec(BC, QM),
                  pl.BlockSpec(cw1.shape, lambda i: (0, 0)),
                  pl.BlockSpec((1, H1), lambda i: (0, 0)),
                  pl.BlockSpec(cw2.shape, lambda i: (0, 0)),
                  pl.BlockSpec((1, cw2.shape[1]), lambda i: (0, 0))],
        out_specs=[pl.BlockSpec((2, BC, QM), lambda i: (0, i, 0)),
                   _row_spec(BC, D),
                   pl.BlockSpec((1, D), lambda i: (0, 0)),
                   pl.BlockSpec((1, D), lambda i: (0, 0))],
        out_shape=[jax.ShapeDtypeStruct((2, NC, QM), f32),
                   jax.ShapeDtypeStruct((NC, D), jnp.bfloat16),
                   jax.ShapeDtypeStruct((1, D), f32),
                   jax.ShapeDtypeStruct((1, D), f32)],
    )(clause_x, csum, cw1, cb1[None, :], cw2, cb2[None, :])
    gcat = g2.reshape(2 * NC, QM)

    p2_rounds = 2
    p2_out_rows = p2_rounds * CH
    sc_pass2 = _make_sc_pass(row_split=False, src_off_per_core=NC,
                             n_rounds=p2_rounds, out_rows=p2_out_rows)
    acc2 = sc_pass2(gcat, ce2d, le2d, zeros_chunk)
    g_lit = acc2[0, :2 * NV]
    vl_pre = acc2[1, :2 * NV]

    new_clause_x = pl.pallas_call(
        functools.partial(_pairnorm_apply_kernel, n=float(NC)),
        grid=(NC // BC,),
        in_specs=[_row_spec(BC, D), _row_spec(BC, D),
                  pl.BlockSpec((1, D), lambda i: (0, 0)),
                  pl.BlockSpec((1, D), lambda i: (0, 0))],
        out_specs=_row_spec(BC, D),
        out_shape=jax.ShapeDtypeStruct((NC, D), f32),
    )(cval, clause_x, ccolsum, csq)

    ldw64 = jnp.repeat(literal_degree_weight.reshape(NV, 2), QM, axis=1)
    vl64 = vl_pre.reshape(NV, 2 * QM)
    dw32 = jnp.repeat(degree_weight, QM, axis=1)
    BU = 1000
    H2 = uw1.shape[1]
    uval, ucolsum, usq = pl.pallas_call(
        _update_kernel,
        grid=(NV // BU,),
        in_specs=[_row_spec(BU, QM), _row_spec(BU, QM), _row_spec(BU, QM),
                  _row_spec(BU, QM), _row_spec(BU, D), _row_spec(BU, 2 * QM),
                  _row_spec(BU, 2 * QM),
                  pl.BlockSpec(uw1.shape, lambda i: (0, 0)),
                  pl.BlockSpec((1, H2), lambda i: (0, 0)),
                  pl.BlockSpec(uw2.shape, lambda i: (0, 0)),
                  pl.BlockSpec((1, H2), lambda i: (0, 0)),
                  pl.BlockSpec(uw3.shape, lambda i: (0, 0)),
                  pl.BlockSpec((1, D), lambda i: (0, 0))],
        out_specs=[_row_spec(BU, D),
                   pl.BlockSpec((1, D), lambda i: (0, 0)),
                   pl.BlockSpec((1, D), lambda i: (0, 0))],
        out_shape=[jax.ShapeDtypeStruct((NV, D), f32),
                   jax.ShapeDtypeStruct((1, D), f32),
                   jax.ShapeDtypeStruct((1, D), f32)],
    )(q, g_lit[:NV], g_lit[NV:], dw32, variable_x, vl64, ldw64,
      uw1, ub1[None, :], uw2, ub2[None, :], uw3, ub3[None, :])

    new_variable_x = pl.pallas_call(
        functools.partial(_pairnorm_apply_kernel, n=float(NV)),
        grid=(NV // BU,),
        in_specs=[_row_spec(BU, D), _row_spec(BU, D),
                  pl.BlockSpec((1, D), lambda i: (0, 0)),
                  pl.BlockSpec((1, D), lambda i: (0, 0))],
        out_specs=_row_spec(BU, D),
        out_shape=jax.ShapeDtypeStruct((NV, D), f32),
    )(uval, variable_x, ucolsum, usq)

    return new_variable_x, new_clause_x


# speedup vs baseline: 1.0915x; 1.0790x over previous
import functools

import jax
import jax.numpy as jnp
from jax import lax
from jax.experimental import pallas as pl
from jax.experimental.pallas import tpu as pltpu
from jax.experimental.pallas import tpu_sc as plsc

NV = 50000
NC = 200000
NE = 800000
D = 128
QM = 32

NE_PAD = 819200
IDX_ROWS = NE_PAD // 128
ROWS_PER_TILE = IDX_ROWS // 16
KGRP = 2
NGRP = ROWS_PER_TILE // KGRP
CH = 53504
CSLOTS = KGRP * 128 + 16
NSENT = 128
W = 32


def _take16(x, idx):
    return lax.gather(
        x, idx[:, None],
        dimension_numbers=lax.GatherDimensionNumbers(
            offset_dims=(), collapsed_slice_dims=(0,), start_index_map=(0,)),
        slice_sizes=(1,), mode=lax.GatherScatterMode.PROMISE_IN_BOUNDS)


def _make_sc_pass(*, row_split, src_off_per_core, n_rounds, out_rows):
    mesh = plsc.VectorSubcoreMesh(core_axis_name="c", subcore_axis_name="s")
    chunk_rows = CH + NSENT
    zrows_per_tile = chunk_rows // 16
    crows_per_tile = CH // 16

    def body(table_hbm, src2d_hbm, dst2d_hbm, zeros_hbm, out_hbm,
             chunk, sidx_a, didx_a, csi_a, csl_a, csl2_a, rows_a,
             sidx_b, didx_b, csi_b, csl_b, csl2_b, rows_b,
             sem_i, sem_ga, sem_gb, sem_sa, sem_sb):
        c = lax.axis_index("c")
        s = lax.axis_index("s")
        lane = lax.iota(jnp.int32, 16)
        tile_row0 = s * ROWS_PER_TILE
        bufs = ((sidx_a, didx_a, csi_a, csl_a, csl2_a, rows_a, sem_ga, sem_sa),
                (sidx_b, didx_b, csi_b, csl_b, csl2_b, rows_b, sem_gb, sem_sb))

        def prep(g, buf, lo):
            sidx, didx, csi_f, csl_f, csl2, _, _, _ = buf
            e0 = (tile_row0 + g * KGRP) * 128
            cp_s = pltpu.async_copy(src2d_hbm.at[pl.ds(e0, KGRP * 128)], sidx, sem_i)
            cp_d = pltpu.async_copy(dst2d_hbm.at[pl.ds(e0, KGRP * 128)], didx, sem_i)
            cp_s.wait()
            cp_d.wait()
            cnt = jnp.int32(0)
            for j in range(KGRP):
                for l in range(8):
                    dv = didx[pl.ds((j * 8 + l) * 16, 16)]
                    sv = sidx[pl.ds((j * 8 + l) * 16, 16)]
                    if not row_split and src_off_per_core:
                        sv = sv + c * src_off_per_core
                    m = (dv >= lo) & (dv < lo + CH)
                    key = jnp.where(m, lane, 16 + lane)
                    dummy = s * 1024 + (j * 8 + l) * 16 + lane
                    sent = CH + ((lane + 16 * (j * 8 + l)) % NSENT)
                    perm = plsc.sort_key_val(key, lane)[1]
                    csi_f[pl.ds(cnt, 16)] = _take16(jnp.where(m, sv, dummy), perm)
                    csl_f[pl.ds(cnt, 16)] = _take16(jnp.where(m, dv - lo, sent), perm)
                    cnt = cnt + jnp.sum(m.astype(jnp.int32))
            ceilb = ((cnt + 127) // 128) * 128
            for i in range(8):
                @pl.when(cnt + 16 * i < ceilb)
                def _(i=i):
                    csi_f[pl.ds(cnt + 16 * i, 16)] = s * 1024 + 16 * i + lane
                    csl_f[pl.ds(cnt + 16 * i, 16)] = CH + ((lane + 16 * i) % NSENT)
            for j in range(KGRP):
                for i in range(8):
                    csl2[j, pl.ds(i * 16, 16)] = csl_f[pl.ds(j * 128 + i * 16, 16)]
            return cnt

        def fire_gather(buf, cnt):
            _, _, csi_f, _, _, rows, sem_g, _ = buf
            for j in range(KGRP):
                @pl.when(cnt > j * 128)
                def _(j=j):
                    pltpu.async_copy(table_hbm.at[csi_f.at[pl.ds(j * 128, 128)]],
                                     rows.at[pl.ds(j * 128, 128)], sem_g)

        def drain_gather(buf, cnt):
            _, _, csi_f, _, _, rows, sem_g, _ = buf
            for j in range(KGRP):
                @pl.when(cnt > j * 128)
                def _(j=j):
                    pltpu.make_async_copy(
                        table_hbm.at[csi_f.at[pl.ds(j * 128, 128)]],
                        rows.at[pl.ds(j * 128, 128)], sem_g).wait()

        def fire_scatter(buf, cnt):
            _, _, _, _, csl2, rows, _, sem_s = buf
            for j in range(KGRP):
                @pl.when(cnt > j * 128)
                def _(j=j):
                    pltpu.async_copy(rows.at[pl.ds(j * 128, 128)],
                                     chunk.at[csl2.at[j]], sem_s, add=True)

        def drain_scatter(buf, cnt):
            _, _, _, _, csl2, rows, _, sem_s = buf
            for j in range(KGRP):
                @pl.when(cnt > j * 128)
                def _(j=j):
                    pltpu.make_async_copy(rows.at[pl.ds(j * 128, 128)],
                                          chunk.at[csl2.at[j]], sem_s).wait()

        for r in range(n_rounds):
            if row_split:
                lo = (r * 2 + c) * CH
                out_base = lo
            else:
                lo = r * CH
                out_base = r * CH
            pltpu.sync_copy(zeros_hbm.at[pl.ds(s * zrows_per_tile, zrows_per_tile)],
                            chunk.at[pl.ds(s * zrows_per_tile, zrows_per_tile)])
            plsc.subcore_barrier()

            cnt_a0 = prep(0, bufs[0], lo)
            fire_gather(bufs[0], cnt_a0)

            def pair(t, cnt_a):
                a, b = bufs
                cnt_b = prep(2 * t + 1, b, lo)
                drain_gather(a, cnt_a)
                fire_scatter(a, cnt_a)
                fire_gather(b, cnt_b)
                drain_scatter(a, cnt_a)
                g_next = jnp.where(2 * t + 2 < NGRP, 2 * t + 2, 0)
                cnt_a2 = prep(g_next, a, lo)
                drain_gather(b, cnt_b)
                fire_scatter(b, cnt_b)
                fire_gather(a, cnt_a2)
                drain_scatter(b, cnt_b)
                return cnt_a2

            cnt_fin = lax.fori_loop(0, NGRP // 2, pair, cnt_a0)
            drain_gather(bufs[0], cnt_fin)
            plsc.subcore_barrier()
            if row_split:
                pltpu.sync_copy(chunk.at[pl.ds(s * crows_per_tile, crows_per_tile)],
                                out_hbm.at[pl.ds(out_base + s * crows_per_tile,
                                                 crows_per_tile)])
            else:
                pltpu.sync_copy(chunk.at[pl.ds(s * crows_per_tile, crows_per_tile)],
                                out_hbm.at[c].at[pl.ds(out_base + s * crows_per_tile,
                                                       crows_per_tile)])
            plsc.subcore_barrier()

    if row_split:
        out_type = jax.ShapeDtypeStruct((out_rows, W), jnp.float32)
    else:
        out_type = jax.ShapeDtypeStruct((2, out_rows, W), jnp.float32)

    return pl.kernel(
        body,
        out_type=out_type,
        mesh=mesh,
        compiler_params=pltpu.CompilerParams(use_tc_tiling_on_sc=False,
                                             needs_layout_passes=False),
        scratch_types=(
            [pltpu.VMEM_SHARED((chunk_rows, W), jnp.float32)]
            + 2 * [pltpu.VMEM((KGRP * 128,), jnp.int32),
                   pltpu.VMEM((KGRP * 128,), jnp.int32),
                   pltpu.VMEM((CSLOTS,), jnp.int32),
                   pltpu.VMEM((CSLOTS,), jnp.int32),
                   pltpu.VMEM((KGRP, 128), jnp.int32),
                   pltpu.VMEM((KGRP * 128, W), jnp.float32)]
            + 5 * [pltpu.SemaphoreType.DMA]
        ),
    )


def _softplus(x):
    return jnp.maximum(x, 0.0) + jnp.log1p(jnp.exp(-jnp.abs(x)))


def _q_kernel(vx_ref, nz_ref, w1a_ref, w1b_ref, b1_ref, w2_ref, b2_ref,
              q_ref, lit_ref):
    h = (jnp.dot(vx_ref[...], w1a_ref[...], preferred_element_type=jnp.float32)
         + jnp.dot(nz_ref[...], w1b_ref[...], preferred_element_type=jnp.float32)
         + b1_ref[...])
    h = jnp.maximum(h, 0.0)
    q = jnp.dot(h, w2_ref[...], preferred_element_type=jnp.float32) + b2_ref[...]
    q_ref[...] = q
    lit_ref[0, :, :] = _softplus(q)
    lit_ref[1, :, :] = _softplus(-q)


def _clause_kernel(cx_ref, csum_ref, w1_ref, b1_ref, w2_ref, b2_ref,
                   g_ref, val_ref, colsum_ref, sq_ref):
    i = pl.program_id(0)
    cl = jnp.exp(-csum_ref[...])
    unit = jnp.concatenate([cx_ref[...], 4.0 * cl], axis=-1)
    h = jnp.dot(unit.astype(jnp.bfloat16), w1_ref[...].astype(jnp.bfloat16),
                preferred_element_type=jnp.float32) + b1_ref[...]
    h = jnp.maximum(h, 0.0)
    d = jnp.dot(h.astype(jnp.bfloat16), w2_ref[...].astype(jnp.bfloat16),
                preferred_element_type=jnp.float32) + b2_ref[...]
    g_ref[0, :, :] = -cl
    g_ref[1, :, :] = d[:, :QM]
    val = d[:, QM:]
    val_ref[...] = val.astype(val_ref.dtype)
    bs = jnp.sum(val, axis=0, keepdims=True)
    bq = jnp.sum(val * val, axis=0, keepdims=True)

    @pl.when(i == 0)
    def _():
        colsum_ref[...] = bs
        sq_ref[...] = bq

    @pl.when(i != 0)
    def _():
        colsum_ref[...] += bs
        sq_ref[...] += bq


def _pairnorm_apply_kernel(val_ref, res_ref, colsum_ref, sq_ref, o_ref, *, n):
    mu = colsum_ref[...] / n
    var = jnp.sum(sq_ref[...]) / (n * D) - jnp.sum(mu * mu) / D
    scale = lax.rsqrt(var + 1e-6)
    o_ref[...] = (val_ref[...].astype(jnp.float32) - mu) * scale + 0.1 * res_ref[...]


def _update_kernel(q_ref, glt_ref, glb_ref, dw_ref, vx_ref, vl_ref, ldw_ref,
                   w1_ref, b1_ref, w2_ref, b2_ref, w3_ref, b3_ref,
                   val_ref, colsum_ref, sq_ref):
    i = pl.program_id(0)
    q = q_ref[...]
    sig_p = jax.nn.sigmoid(q)
    sig_n = jax.nn.sigmoid(-q)
    vg = (sig_p * glt_ref[0] - sig_n * glb_ref[0]) * dw_ref[...]
    b = vl_ref.shape[0]
    ldw = ldw_ref[...]
    ldw64 = jnp.concatenate([jnp.broadcast_to(ldw[:, 0:1], (b, QM)),
                             jnp.broadcast_to(ldw[:, 1:2], (b, QM))], axis=1)
    vl = vl_ref[...] * ldw64
    unit = jnp.concatenate([vg, vx_ref[...], vl], axis=-1)
    h = jnp.dot(unit.astype(jnp.bfloat16), w1_ref[...].astype(jnp.bfloat16),
                preferred_element_type=jnp.float32) + b1_ref[...]
    h = jnp.maximum(h, 0.0)
    h = jnp.dot(h.astype(jnp.bfloat16), w2_ref[...].astype(jnp.bfloat16),
                preferred_element_type=jnp.float32) + b2_ref[...]
    h = jnp.maximum(h, 0.0)
    val = jnp.dot(h.astype(jnp.bfloat16), w3_ref[...].astype(jnp.bfloat16),
                  preferred_element_type=jnp.float32) + b3_ref[...]
    val_ref[...] = val
    bs = jnp.sum(val, axis=0, keepdims=True)
    bq = jnp.sum(val * val, axis=0, keepdims=True)

    @pl.when(i == 0)
    def _():
        colsum_ref[...] = bs
        sq_ref[...] = bq

    @pl.when(i != 0)
    def _():
        colsum_ref[...] += bs
        sq_ref[...] += bq


def _row_spec(b, w):
    return pl.BlockSpec((b, w), lambda i: (i, 0))


def _full_spec(shape):
    return pl.BlockSpec(shape, lambda i: tuple(0 for _ in shape))


def kernel(variable_x, clause_x, noise, degree_weight, literal_degree_weight,
           clause_edge_index, literal_edge_index,
           query_params, clause_params, update_params):
    f32 = jnp.float32
    qw1, qb1, qw2, qb2 = query_params
    cw1, cb1, cw2, cb2 = clause_params
    uw1, ub1, uw2, ub2, uw3, ub3 = update_params

    BQ = 1000
    q, lit2 = pl.pallas_call(
        _q_kernel,
        grid=(NV // BQ,),
        in_specs=[_row_spec(BQ, D), _row_spec(BQ, 4),
                  pl.BlockSpec((D, qw1.shape[1]), lambda i: (0, 0)),
                  pl.BlockSpec((4, qw1.shape[1]), lambda i: (0, 0)),
                  pl.BlockSpec((1, qb1.shape[0]), lambda i: (0, 0)),
                  pl.BlockSpec(qw2.shape, lambda i: (0, 0)),
                  pl.BlockSpec((1, QM), lambda i: (0, 0))],
        out_specs=[_row_spec(BQ, QM),
                   pl.BlockSpec((2, BQ, QM), lambda i: (0, i, 0))],
        out_shape=[jax.ShapeDtypeStruct((NV, QM), f32),
                   jax.ShapeDtypeStruct((2, NV, QM), f32)],
    )(variable_x, noise, qw1[:D], qw1[D:], qb1[None, :], qw2, qb2[None, :])
    lit = lit2.reshape(2 * NV, QM)

    pad = NE_PAD - NE
    le2d = jnp.concatenate([literal_edge_index,
                            jnp.full((pad,), -1, jnp.int32)])
    ce2d = jnp.concatenate([clause_edge_index,
                            jnp.full((pad,), -1, jnp.int32)])
    zeros_chunk = jnp.zeros((CH + NSENT, W), f32)

    p1_rounds = 2
    p1_out_rows = p1_rounds * 2 * CH
    sc_pass1 = _make_sc_pass(row_split=True, src_off_per_core=0,
                             n_rounds=p1_rounds, out_rows=p1_out_rows)
    csum = sc_pass1(lit, le2d, ce2d, zeros_chunk)

    BC = 2000
    H1 = cw1.shape[1]
    g2, cval, ccolsum, csq = pl.pallas_call(
        _clause_kernel,
        grid=(NC // BC,),
        in_specs=[_row_spec(BC, D), _row_spec(BC, QM),
                  pl.BlockSpec(cw1.shape, lambda i: (0, 0)),
                  pl.BlockSpec((1, H1), lambda i: (0, 0)),
                  pl.BlockSpec(cw2.shape, lambda i: (0, 0)),
                  pl.BlockSpec((1, cw2.shape[1]), lambda i: (0, 0))],
        out_specs=[pl.BlockSpec((2, BC, QM), lambda i: (0, i, 0)),
                   _row_spec(BC, D),
                   pl.BlockSpec((1, D), lambda i: (0, 0)),
                   pl.BlockSpec((1, D), lambda i: (0, 0))],
        out_shape=[jax.ShapeDtypeStruct((2, NC, QM), f32),
                   jax.ShapeDtypeStruct((NC, D), jnp.bfloat16),
                   jax.ShapeDtypeStruct((1, D), f32),
                   jax.ShapeDtypeStruct((1, D), f32)],
    )(clause_x, csum, cw1, cb1[None, :], cw2, cb2[None, :])
    gcat = g2.reshape(2 * NC, QM)

    p2_rounds = 2
    p2_out_rows = p2_rounds * CH
    sc_pass2 = _make_sc_pass(row_split=False, src_off_per_core=NC,
                             n_rounds=p2_rounds, out_rows=p2_out_rows)
    acc2 = sc_pass2(gcat, ce2d, le2d, zeros_chunk)
    vl_pre = acc2[1, :2 * NV]

    new_clause_x = pl.pallas_call(
        functools.partial(_pairnorm_apply_kernel, n=float(NC)),
        grid=(NC // BC,),
        in_specs=[_row_spec(BC, D), _row_spec(BC, D),
                  pl.BlockSpec((1, D), lambda i: (0, 0)),
                  pl.BlockSpec((1, D), lambda i: (0, 0))],
        out_specs=_row_spec(BC, D),
        out_shape=jax.ShapeDtypeStruct((NC, D), f32),
    )(cval, clause_x, ccolsum, csq)

    ldw2 = literal_degree_weight.reshape(NV, 2)
    vl64 = vl_pre.reshape(NV, 2 * QM)
    BU = 1000
    H2 = uw1.shape[1]
    uval, ucolsum, usq = pl.pallas_call(
        _update_kernel,
        grid=(NV // BU,),
        in_specs=[_row_spec(BU, QM),
                  pl.BlockSpec((1, BU, QM), lambda i: (0, i, 0)),
                  pl.BlockSpec((1, BU, QM), lambda i: (0, NV // BU + i, 0)),
                  pl.BlockSpec((BU, 1), lambda i: (i, 0)), _row_spec(BU, D),
                  _row_spec(BU, 2 * QM),
                  pl.BlockSpec((BU, 2), lambda i: (i, 0)),
                  pl.BlockSpec(uw1.shape, lambda i: (0, 0)),
                  pl.BlockSpec((1, H2), lambda i: (0, 0)),
                  pl.BlockSpec(uw2.shape, lambda i: (0, 0)),
                  pl.BlockSpec((1, H2), lambda i: (0, 0)),
                  pl.BlockSpec(uw3.shape, lambda i: (0, 0)),
                  pl.BlockSpec((1, D), lambda i: (0, 0))],
        out_specs=[_row_spec(BU, D),
                   pl.BlockSpec((1, D), lambda i: (0, 0)),
                   pl.BlockSpec((1, D), lambda i: (0, 0))],
        out_shape=[jax.ShapeDtypeStruct((NV, D), f32),
                   jax.ShapeDtypeStruct((1, D), f32),
                   jax.ShapeDtypeStruct((1, D), f32)],
    )(q, acc2, acc2, degree_weight, variable_x, vl64, ldw2,
      uw1, ub1[None, :], uw2, ub2[None, :], uw3, ub3[None, :])

    new_variable_x = pl.pallas_call(
        functools.partial(_pairnorm_apply_kernel, n=float(NV)),
        grid=(NV // BU,),
        in_specs=[_row_spec(BU, D), _row_spec(BU, D),
                  pl.BlockSpec((1, D), lambda i: (0, 0)),
                  pl.BlockSpec((1, D), lambda i: (0, 0))],
        out_specs=_row_spec(BU, D),
        out_shape=jax.ShapeDtypeStruct((NV, D), f32),
    )(uval, variable_x, ucolsum, usq)

    return new_variable_x, new_clause_x
